# baseline (speedup 1.0000x reference)
import functools

import jax
import jax.numpy as jnp
from jax import lax
from jax.experimental import pallas as pl
from jax.experimental.pallas import tpu as pltpu
from jax.experimental.pallas import tpu_sc as plsc

D = 1024
_info = plsc.get_sparse_core_info()
_NC, _NS = _info.num_cores, _info.num_subcores
_NW = _NC * _NS

_CH = 32
_RB = _CH // 16


def _sc_body(seg_hbm, table_hbm, out_hbm, idx_v, buf0, buf1, table_v,
             ssem0, ssem1):
    b = seg_hbm.shape[0]
    b_per_w = b // _NW
    sid = lax.axis_index("s")
    cid = lax.axis_index("c")
    wid = sid * _NC + cid
    base = wid * b_per_w

    pltpu.sync_copy(table_hbm, table_v)
    pltpu.sync_copy(seg_hbm.at[pl.ds(base, b_per_w)], idx_v)

    bufs = (buf0, buf1)
    ssems = (ssem0, ssem1)
    n = b_per_w // _CH
    lanes = lax.iota(jnp.int32, 16)
    dst0 = [(lanes + rb * 16) * D for rb in range(_RB)]

    def build(i, buf):
        src0 = [idx_v[pl.ds(i * _CH + rb * 16, 16)] * D for rb in range(_RB)]

        def col(c, carry):
            cvec = jnp.full((16,), 0, jnp.int32) + c
            for rb in range(_RB):
                val = plsc.load_gather(table_v, [src0[rb] + cvec])
                plsc.store_scatter(buf, [dst0[rb] + cvec], val)
            return carry

        lax.fori_loop(0, D, col, jnp.int32(0), unroll=8)

    def scatter(i):
        j = i & 1
        return pltpu.async_copy(
            bufs[j], out_hbm.at[pl.ds((base + i * _CH) * D, _CH * D)],
            ssems[j])

    descs = [None, None]
    for i in range(n):
        j = i & 1
        if descs[j] is not None:
            descs[j].wait()
        build(i, bufs[j])
        descs[j] = scatter(i)
    descs[0].wait()
    descs[1].wait()


@jax.jit
def _sc_lookup(seg_flat, table_flat):
    b = seg_flat.shape[0]
    b_per_w = b // _NW
    mesh = plsc.VectorSubcoreMesh(core_axis_name="c", subcore_axis_name="s")
    return pl.kernel(
        _sc_body,
        out_type=jax.ShapeDtypeStruct((b * D,), jnp.float32),
        mesh=mesh,
        scratch_types=[
            pltpu.VMEM((b_per_w,), jnp.int32),
            pltpu.VMEM((_CH * D,), jnp.float32),
            pltpu.VMEM((_CH * D,), jnp.float32),
            pltpu.VMEM((2 * D,), jnp.float32),
            pltpu.SemaphoreType.DMA,
            pltpu.SemaphoreType.DMA,
        ],
        compiler_params=pltpu.CompilerParams(needs_layout_passes=False),
    )(seg_flat, table_flat)


def kernel(segments, table):
    bsz, seq = segments.shape
    seg_flat = segments.reshape(bsz * seq).astype(jnp.int32)
    out = _sc_lookup(seg_flat, table.reshape(2 * D))
    return out.reshape(bsz, seq, D)


# speedup vs baseline: 6.0147x; 6.0147x over previous
import functools

import jax
import jax.numpy as jnp
from jax import lax
from jax.experimental import pallas as pl
from jax.experimental.pallas import tpu as pltpu
from jax.experimental.pallas import tpu_sc as plsc

D = 1024
_info = plsc.get_sparse_core_info()
_NC, _NS = _info.num_cores, _info.num_subcores
_NW = _NC * _NS

_G = 16
_NBUF = 4


def _sc_body(seg_hbm, table_hbm, out_hbm, idx_v, buf_all,
             dp0, dp1, dp2, dp3, sem0, sem1, sem2, sem3):
    b = seg_hbm.shape[0]
    b_per_w = b // _NW
    sid = lax.axis_index("s")
    cid = lax.axis_index("c")
    wid = sid * _NC + cid
    base = wid * b_per_w

    pltpu.sync_copy(seg_hbm.at[pl.ds(base, b_per_w)], idx_v)

    for r in range(16):
        pltpu.sync_copy(table_hbm.at[pl.ds(0, 1)], buf_all.at[pl.ds(r, 1)])
        pltpu.sync_copy(table_hbm.at[pl.ds(1, 1)], buf_all.at[pl.ds(16 + r, 1)])

    dps = (dp0, dp1, dp2, dp3)
    sems = (sem0, sem1, sem2, sem3)
    lanes = lax.iota(jnp.int32, 16)
    n = b_per_w // _G

    descs = [None] * _NBUF

    for g in range(n):
        j = g % _NBUF
        if descs[j] is not None:
            descs[j].wait()
        seg16 = idx_v[pl.ds(g * _G, 16)]
        pos16 = jnp.full((16,), 0, jnp.int32) + (base + g * _G) + lanes
        _, perm = plsc.sort_key_val(seg16, pos16)
        dps[j][...] = perm
        k1 = lax.reduce_sum(seg16, axes=(0,))
        descs[j] = pltpu.async_copy(
            buf_all.at[pl.ds(k1, 16)], out_hbm.at[dps[j]], sems[j])
    for j in range(_NBUF):
        descs[j].wait()


@jax.jit
def _sc_lookup(seg_flat, table):
    b = seg_flat.shape[0]
    b_per_w = b // _NW
    mesh = plsc.VectorSubcoreMesh(core_axis_name="c", subcore_axis_name="s")
    return pl.kernel(
        _sc_body,
        out_type=jax.ShapeDtypeStruct((b, D), jnp.float32),
        mesh=mesh,
        scratch_types=[
            pltpu.VMEM((b_per_w,), jnp.int32),
            pltpu.VMEM((32, D), jnp.float32),
            pltpu.VMEM((16,), jnp.int32),
            pltpu.VMEM((16,), jnp.int32),
            pltpu.VMEM((16,), jnp.int32),
            pltpu.VMEM((16,), jnp.int32),
            pltpu.SemaphoreType.DMA,
            pltpu.SemaphoreType.DMA,
            pltpu.SemaphoreType.DMA,
            pltpu.SemaphoreType.DMA,
        ],
        compiler_params=pltpu.CompilerParams(
            use_tc_tiling_on_sc=False, needs_layout_passes=False),
    )(seg_flat, table)


def kernel(segments, table):
    bsz, seq = segments.shape
    seg_flat = segments.reshape(bsz * seq).astype(jnp.int32)
    out = _sc_lookup(seg_flat, table)
    return out.reshape(bsz, seq, D)
